# baseline (speedup 1.0000x reference)
import functools

import jax
import jax.numpy as jnp
from jax import lax
from jax.experimental import pallas as pl
from jax.experimental.pallas import tpu as pltpu
from jax.experimental.pallas import tpu_sc as plsc

DIM = 768
D_FF = 256
N_EXPERTS = 64
N_SHARED = 2
N_ROUTED = N_EXPERTS - N_SHARED
SHARED_FF = N_SHARED * D_FF
T = 4 * 2048

EP = 64
BT_R = 512
NB_R = T // BT_R
BT_G = 128
NT = (T + N_ROUTED * (BT_G - 1) + BT_G - 1) // BT_G
NTP = 128
P = NT * BT_G
BT_S = 512
NB_S = T // BT_S

_NC = 2
_NS = 16
_NW = _NC * _NS
_TPW = T // _NW
_CH = 128
_NCH = _TPW // _CH
_L = 16


def _sc_scatter_body(pos_hbm, x_hbm, xs_hbm, pos_v, rows_v, sem):
    wid = lax.axis_index("s") * _NC + lax.axis_index("c")
    base = wid * _TPW
    for c in range(_NCH):
        off = base + c * _CH
        pltpu.sync_copy(pos_hbm.at[pl.ds(off, _CH)], pos_v)
        pltpu.sync_copy(x_hbm.at[pl.ds(off, _CH)], rows_v)
        pltpu.async_copy(rows_v, xs_hbm.at[pos_v], sem).wait()


def _sc_gather_body(pos_hbm, src_hbm, out_hbm, pos_v, rows_v, sem):
    wid = lax.axis_index("s") * _NC + lax.axis_index("c")
    base = wid * _TPW
    for c in range(_NCH):
        off = base + c * _CH
        pltpu.sync_copy(pos_hbm.at[pl.ds(off, _CH)], pos_v)
        pltpu.async_copy(src_hbm.at[pos_v], rows_v, sem).wait()
        pltpu.sync_copy(rows_v, out_hbm.at[pl.ds(off, _CH)])


@functools.lru_cache(maxsize=1)
def _sc_kernels():
    scratch = [
        pltpu.VMEM((_CH,), jnp.int32),
        pltpu.VMEM((_CH, DIM), jnp.float32),
        pltpu.SemaphoreType.DMA,
    ]
    mesh = plsc.VectorSubcoreMesh(core_axis_name="c", subcore_axis_name="s")
    scatter = functools.partial(
        pl.kernel, mesh=mesh,
        out_type=jax.ShapeDtypeStruct((P, DIM), jnp.float32),
        scratch_types=scratch,
    )(_sc_scatter_body)
    gather = functools.partial(
        pl.kernel, mesh=mesh,
        out_type=jax.ShapeDtypeStruct((T, DIM), jnp.float32),
        scratch_types=scratch,
    )(_sc_gather_body)
    return scatter, gather


def _sc_scatter(pos, xf):
    return _sc_kernels()[0](pos, xf)


def _sc_gather(pos, src):
    return _sc_kernels()[1](pos, src)


def _router_body(x_ref, wg_ref, eid_ref, gate_ref, rank_ref, starts_ref,
                 te_ref, cnt):
    pid = pl.program_id(0)

    @pl.when(pid == 0)
    def _():
        cnt[...] = jnp.zeros_like(cnt)

    x = x_ref[...]
    logits = jnp.dot(x, wg_ref[...], preferred_element_type=jnp.float32)
    col = jax.lax.broadcasted_iota(jnp.int32, logits.shape, 1)
    logits = jnp.where(col < N_ROUTED, logits, -1e30)
    m = jnp.max(logits, axis=1, keepdims=True)
    ssum = jnp.sum(jnp.exp(logits - m), axis=1)
    eid = jnp.min(jnp.where(logits == m, col, EP), axis=1)
    onehot = (col == eid[:, None]).astype(jnp.float32)
    row_i = jax.lax.broadcasted_iota(jnp.int32, (BT_R, BT_R), 0)
    col_i = jax.lax.broadcasted_iota(jnp.int32, (BT_R, BT_R), 1)
    tri = (row_i >= col_i).astype(jnp.float32)
    csum = jnp.dot(tri, onehot, preferred_element_type=jnp.float32)
    rank_in_blk = jnp.sum(onehot * csum, axis=1) - 1.0
    prev = jnp.sum(onehot * cnt[0:1, :], axis=1)
    eid_ref[0, 0, :] = eid
    gate_ref[0, 0, :] = 1.0 / ssum
    rank_ref[0, 0, :] = (prev + rank_in_blk).astype(jnp.int32)
    cnt[0:1, :] = cnt[0:1, :] + csum[BT_R - 1:BT_R, :]

    @pl.when(pid == NB_R - 1)
    def _():
        c = cnt[0:1, :]
        padded = jnp.floor((c + (BT_G - 1)) * (1.0 / BT_G)) * BT_G
        r64 = jax.lax.broadcasted_iota(jnp.int32, (EP, EP), 0)
        c64 = jax.lax.broadcasted_iota(jnp.int32, (EP, EP), 1)
        excl = (r64 < c64).astype(jnp.float32)
        incl = (r64 <= c64).astype(jnp.float32)
        starts = jnp.dot(padded, excl, preferred_element_type=jnp.float32)
        tiles_cum = jnp.dot(padded * (1.0 / BT_G), incl,
                            preferred_element_type=jnp.float32)
        starts_ref[...] = jnp.broadcast_to(starts, (8, EP))
        tc_b = jax.lax.broadcast_in_dim(tiles_cum.reshape(EP), (EP, NTP), (0,))
        e_b = jax.lax.broadcasted_iota(jnp.int32, (EP, NTP), 0)
        j_b = jax.lax.broadcasted_iota(jnp.int32, (EP, NTP), 1)
        hit = jnp.where((tc_b <= j_b.astype(jnp.float32)) & (e_b < N_ROUTED),
                        1.0, 0.0)
        te = jnp.minimum(jnp.sum(hit, axis=0), float(N_ROUTED - 1))
        te_ref[...] = jnp.broadcast_to(te[None, :], (8, NTP)).astype(jnp.int32)


def _moe_body(te_ref, xs_ref, w1_ref, w3_ref, w2_ref, out_ref):
    del te_ref
    x = xs_ref[...]
    w1 = w1_ref[0]
    w3 = w3_ref[0]
    w2 = w2_ref[0]
    dn = (((1,), (1,)), ((), ()))
    a = jax.lax.dot_general(x, w1, dn, preferred_element_type=jnp.float32)
    b = jax.lax.dot_general(x, w3, dn, preferred_element_type=jnp.float32)
    h = a * jax.lax.logistic(a) * b
    out_ref[...] = jax.lax.dot_general(h, w2, dn,
                                       preferred_element_type=jnp.float32)


def _shared_body(x_ref, w1_ref, w3_ref, w2_ref, yg_ref, gw_ref, out_ref):
    x = x_ref[...]
    dn = (((1,), (1,)), ((), ()))
    a = jax.lax.dot_general(x, w1_ref[...], dn,
                            preferred_element_type=jnp.float32)
    b = jax.lax.dot_general(x, w3_ref[...], dn,
                            preferred_element_type=jnp.float32)
    h = a * jax.lax.logistic(a) * b
    z = jax.lax.dot_general(h, w2_ref[...], dn,
                            preferred_element_type=jnp.float32)
    w = gw_ref[0, 0, :]
    out_ref[...] = z + yg_ref[...] * w[:, None]


def kernel(x, Wg, We1, We3, We2, Ws1, Ws3, Ws2):
    shape = x.shape
    xf = x.reshape(T, DIM)
    wg_pad = jnp.pad(Wg, ((0, 0), (0, EP - N_ROUTED)))

    eid3, gate3, rank3, starts, tile_expert = pl.pallas_call(
        _router_body,
        grid=(NB_R,),
        in_specs=[
            pl.BlockSpec((BT_R, DIM), lambda i: (i, 0)),
            pl.BlockSpec((DIM, EP), lambda i: (0, 0)),
        ],
        out_specs=[
            pl.BlockSpec((1, 1, BT_R), lambda i: (i, 0, 0)),
            pl.BlockSpec((1, 1, BT_R), lambda i: (i, 0, 0)),
            pl.BlockSpec((1, 1, BT_R), lambda i: (i, 0, 0)),
            pl.BlockSpec((8, EP), lambda i: (0, 0)),
            pl.BlockSpec((8, NTP), lambda i: (0, 0)),
        ],
        out_shape=[
            jax.ShapeDtypeStruct((NB_R, 1, BT_R), jnp.int32),
            jax.ShapeDtypeStruct((NB_R, 1, BT_R), jnp.float32),
            jax.ShapeDtypeStruct((NB_R, 1, BT_R), jnp.int32),
            jax.ShapeDtypeStruct((8, EP), jnp.float32),
            jax.ShapeDtypeStruct((8, NTP), jnp.int32),
        ],
        scratch_shapes=[pltpu.VMEM((1, EP), jnp.float32)],
    )(xf, wg_pad)

    pos = starts[0, eid3.reshape(T)].astype(jnp.int32) + rank3.reshape(T)

    xs = _sc_scatter(pos, xf)

    out_padded = pl.pallas_call(
        _moe_body,
        grid_spec=pltpu.PrefetchScalarGridSpec(
            num_scalar_prefetch=1,
            grid=(NT,),
            in_specs=[
                pl.BlockSpec((BT_G, DIM), lambda j, te: (j, 0)),
                pl.BlockSpec((1, D_FF, DIM), lambda j, te: (te[0, j], 0, 0)),
                pl.BlockSpec((1, D_FF, DIM), lambda j, te: (te[0, j], 0, 0)),
                pl.BlockSpec((1, DIM, D_FF), lambda j, te: (te[0, j], 0, 0)),
            ],
            out_specs=pl.BlockSpec((BT_G, DIM), lambda j, te: (j, 0)),
        ),
        out_shape=jax.ShapeDtypeStruct((P, DIM), jnp.float32),
    )(tile_expert, xs, We1, We3, We2)

    yg = _sc_gather(pos, out_padded)

    out = pl.pallas_call(
        _shared_body,
        grid=(NB_S,),
        in_specs=[
            pl.BlockSpec((BT_S, DIM), lambda i: (i, 0)),
            pl.BlockSpec((SHARED_FF, DIM), lambda i: (0, 0)),
            pl.BlockSpec((SHARED_FF, DIM), lambda i: (0, 0)),
            pl.BlockSpec((DIM, SHARED_FF), lambda i: (0, 0)),
            pl.BlockSpec((BT_S, DIM), lambda i: (i, 0)),
            pl.BlockSpec((1, 1, BT_S), lambda i: (i, 0, 0)),
        ],
        out_specs=pl.BlockSpec((BT_S, DIM), lambda i: (i, 0)),
        out_shape=jax.ShapeDtypeStruct((T, DIM), jnp.float32),
    )(xf, Ws1, Ws3, Ws2, yg, gate3)

    return out.reshape(shape)


# speedup vs baseline: 1.1370x; 1.1370x over previous
import functools

import jax
import jax.numpy as jnp
from jax import lax
from jax.experimental import pallas as pl
from jax.experimental.pallas import tpu as pltpu
from jax.experimental.pallas import tpu_sc as plsc

DIM = 768
D_FF = 256
N_EXPERTS = 64
N_SHARED = 2
N_ROUTED = N_EXPERTS - N_SHARED
SHARED_FF = N_SHARED * D_FF
T = 4 * 2048

EP = 64
BT_R = 512
NB_R = T // BT_R
BT_G = 128
NT = (T + N_ROUTED * (BT_G - 1) + BT_G - 1) // BT_G
NTP = 128
P = NT * BT_G
BT_S = 512
NB_S = T // BT_S

_NC = 2
_NS = 16
_NW = _NC * _NS
_TPW = T // _NW
_CH = 128
_NCH = _TPW // _CH
_L = 16


def _sc_scatter_body(pos_hbm, x_hbm, xs_hbm, pos_v, rows_v, sem):
    wid = lax.axis_index("s") * _NC + lax.axis_index("c")
    base = wid * _TPW
    for c in range(_NCH):
        off = base + c * _CH
        pltpu.sync_copy(pos_hbm.at[pl.ds(off, _CH)], pos_v)
        pltpu.sync_copy(x_hbm.at[pl.ds(off, _CH)], rows_v)
        pltpu.async_copy(rows_v, xs_hbm.at[pos_v], sem).wait()


def _sc_gather_body(pos_hbm, src_hbm, out_hbm, pos_v, rows_v, sem):
    wid = lax.axis_index("s") * _NC + lax.axis_index("c")
    base = wid * _TPW
    for c in range(_NCH):
        off = base + c * _CH
        pltpu.sync_copy(pos_hbm.at[pl.ds(off, _CH)], pos_v)
        pltpu.async_copy(src_hbm.at[pos_v], rows_v, sem).wait()
        pltpu.sync_copy(rows_v, out_hbm.at[pl.ds(off, _CH)])


@functools.lru_cache(maxsize=1)
def _sc_kernels():
    scratch = [
        pltpu.VMEM((_CH,), jnp.int32),
        pltpu.VMEM((_CH, DIM), jnp.float32),
        pltpu.SemaphoreType.DMA,
    ]
    mesh = plsc.VectorSubcoreMesh(core_axis_name="c", subcore_axis_name="s")
    scatter = functools.partial(
        pl.kernel, mesh=mesh,
        out_type=jax.ShapeDtypeStruct((P, DIM), jnp.float32),
        scratch_types=scratch,
    )(_sc_scatter_body)
    gather = functools.partial(
        pl.kernel, mesh=mesh,
        out_type=jax.ShapeDtypeStruct((T, DIM), jnp.float32),
        scratch_types=scratch,
    )(_sc_gather_body)
    return scatter, gather


def _sc_scatter(pos, xf):
    return _sc_kernels()[0](pos, xf)


def _sc_gather(pos, src):
    return _sc_kernels()[1](pos, src)


def _router_body(x_ref, wg_ref, eid_ref, gate_ref, rank_ref, starts_ref,
                 te_ref, cnt):
    pid = pl.program_id(0)

    @pl.when(pid == 0)
    def _():
        cnt[...] = jnp.zeros_like(cnt)

    x = x_ref[...]
    logits = jnp.dot(x, wg_ref[...], preferred_element_type=jnp.float32)
    col = jax.lax.broadcasted_iota(jnp.int32, logits.shape, 1)
    logits = jnp.where(col < N_ROUTED, logits, -1e30)
    m = jnp.max(logits, axis=1, keepdims=True)
    ssum = jnp.sum(jnp.exp(logits - m), axis=1)
    eid = jnp.min(jnp.where(logits == m, col, EP), axis=1)
    onehot = (col == eid[:, None]).astype(jnp.float32)
    row_i = jax.lax.broadcasted_iota(jnp.int32, (BT_R, BT_R), 0)
    col_i = jax.lax.broadcasted_iota(jnp.int32, (BT_R, BT_R), 1)
    tri = (row_i >= col_i).astype(jnp.float32)
    csum = jnp.dot(tri, onehot, preferred_element_type=jnp.float32)
    rank_in_blk = jnp.sum(onehot * csum, axis=1) - 1.0
    prev = jnp.sum(onehot * cnt[0:1, :], axis=1)
    eid_ref[0, 0, :] = eid
    gate_ref[0, 0, :] = 1.0 / ssum
    rank_ref[0, 0, :] = (prev + rank_in_blk).astype(jnp.int32)
    cnt[0:1, :] = cnt[0:1, :] + csum[BT_R - 1:BT_R, :]

    @pl.when(pid == NB_R - 1)
    def _():
        c = cnt[0:1, :]
        padded = jnp.floor((c + (BT_G - 1)) * (1.0 / BT_G)) * BT_G
        r64 = jax.lax.broadcasted_iota(jnp.int32, (EP, EP), 0)
        c64 = jax.lax.broadcasted_iota(jnp.int32, (EP, EP), 1)
        excl = (r64 < c64).astype(jnp.float32)
        incl = (r64 <= c64).astype(jnp.float32)
        starts = jnp.dot(padded, excl, preferred_element_type=jnp.float32)
        tiles_cum = jnp.dot(padded * (1.0 / BT_G), incl,
                            preferred_element_type=jnp.float32)
        starts_ref[...] = jnp.broadcast_to(starts, (8, EP))
        tc_b = jax.lax.broadcast_in_dim(tiles_cum.reshape(EP), (EP, NTP), (0,))
        e_b = jax.lax.broadcasted_iota(jnp.int32, (EP, NTP), 0)
        j_b = jax.lax.broadcasted_iota(jnp.int32, (EP, NTP), 1)
        hit = jnp.where((tc_b <= j_b.astype(jnp.float32)) & (e_b < N_ROUTED),
                        1.0, 0.0)
        te = jnp.minimum(jnp.sum(hit, axis=0), float(N_ROUTED - 1))
        te_ref[...] = jnp.broadcast_to(te[None, :], (8, NTP)).astype(jnp.int32)


def _pos_body(eid_ref, rank_ref, starts_ref, pos_ref):
    eid = eid_ref[0, 0, :]
    onehot = (jax.lax.broadcasted_iota(jnp.int32, (BT_R, EP), 1)
              == eid[:, None]).astype(jnp.float32)
    st = jnp.sum(onehot * starts_ref[0:1, :], axis=1)
    pos_ref[0, 0, :] = st.astype(jnp.int32) + rank_ref[0, 0, :]


def _moe_body(te_ref, xs_ref, w1_ref, w3_ref, w2_ref, out_ref):
    del te_ref
    x = xs_ref[...]
    w1 = w1_ref[0]
    w3 = w3_ref[0]
    w2 = w2_ref[0]
    dn = (((1,), (1,)), ((), ()))
    a = jax.lax.dot_general(x, w1, dn, preferred_element_type=jnp.float32)
    b = jax.lax.dot_general(x, w3, dn, preferred_element_type=jnp.float32)
    h = a * jax.lax.logistic(a) * b
    out_ref[...] = jax.lax.dot_general(h, w2, dn,
                                       preferred_element_type=jnp.float32)


def _shared_body(x_ref, w1_ref, w3_ref, w2_ref, yg_ref, gw_ref, out_ref):
    x = x_ref[...]
    dn = (((1,), (1,)), ((), ()))
    a = jax.lax.dot_general(x, w1_ref[...], dn,
                            preferred_element_type=jnp.float32)
    b = jax.lax.dot_general(x, w3_ref[...], dn,
                            preferred_element_type=jnp.float32)
    h = a * jax.lax.logistic(a) * b
    z = jax.lax.dot_general(h, w2_ref[...], dn,
                            preferred_element_type=jnp.float32)
    w = gw_ref[0, 0, :]
    out_ref[...] = z + yg_ref[...] * w[:, None]


def kernel(x, Wg, We1, We3, We2, Ws1, Ws3, Ws2):
    shape = x.shape
    xf = x.reshape(T, DIM)
    wg_pad = jnp.pad(Wg, ((0, 0), (0, EP - N_ROUTED)))

    eid3, gate3, rank3, starts, tile_expert = pl.pallas_call(
        _router_body,
        grid=(NB_R,),
        in_specs=[
            pl.BlockSpec((BT_R, DIM), lambda i: (i, 0)),
            pl.BlockSpec((DIM, EP), lambda i: (0, 0)),
        ],
        out_specs=[
            pl.BlockSpec((1, 1, BT_R), lambda i: (i, 0, 0)),
            pl.BlockSpec((1, 1, BT_R), lambda i: (i, 0, 0)),
            pl.BlockSpec((1, 1, BT_R), lambda i: (i, 0, 0)),
            pl.BlockSpec((8, EP), lambda i: (0, 0)),
            pl.BlockSpec((8, NTP), lambda i: (0, 0)),
        ],
        out_shape=[
            jax.ShapeDtypeStruct((NB_R, 1, BT_R), jnp.int32),
            jax.ShapeDtypeStruct((NB_R, 1, BT_R), jnp.float32),
            jax.ShapeDtypeStruct((NB_R, 1, BT_R), jnp.int32),
            jax.ShapeDtypeStruct((8, EP), jnp.float32),
            jax.ShapeDtypeStruct((8, NTP), jnp.int32),
        ],
        scratch_shapes=[pltpu.VMEM((1, EP), jnp.float32)],
    )(xf, wg_pad)

    pos3 = pl.pallas_call(
        _pos_body,
        grid=(NB_R,),
        in_specs=[
            pl.BlockSpec((1, 1, BT_R), lambda i: (i, 0, 0)),
            pl.BlockSpec((1, 1, BT_R), lambda i: (i, 0, 0)),
            pl.BlockSpec((8, EP), lambda i: (0, 0)),
        ],
        out_specs=pl.BlockSpec((1, 1, BT_R), lambda i: (i, 0, 0)),
        out_shape=jax.ShapeDtypeStruct((NB_R, 1, BT_R), jnp.int32),
    )(eid3, rank3, starts)
    pos = pos3.reshape(T)

    xs = _sc_scatter(pos, xf)

    out_padded = pl.pallas_call(
        _moe_body,
        grid_spec=pltpu.PrefetchScalarGridSpec(
            num_scalar_prefetch=1,
            grid=(NT,),
            in_specs=[
                pl.BlockSpec((BT_G, DIM), lambda j, te: (j, 0)),
                pl.BlockSpec((1, D_FF, DIM), lambda j, te: (te[0, j], 0, 0)),
                pl.BlockSpec((1, D_FF, DIM), lambda j, te: (te[0, j], 0, 0)),
                pl.BlockSpec((1, DIM, D_FF), lambda j, te: (te[0, j], 0, 0)),
            ],
            out_specs=pl.BlockSpec((BT_G, DIM), lambda j, te: (j, 0)),
        ),
        out_shape=jax.ShapeDtypeStruct((P, DIM), jnp.float32),
    )(tile_expert, xs, We1, We3, We2)

    yg = _sc_gather(pos, out_padded)

    out = pl.pallas_call(
        _shared_body,
        grid=(NB_S,),
        in_specs=[
            pl.BlockSpec((BT_S, DIM), lambda i: (i, 0)),
            pl.BlockSpec((SHARED_FF, DIM), lambda i: (0, 0)),
            pl.BlockSpec((SHARED_FF, DIM), lambda i: (0, 0)),
            pl.BlockSpec((DIM, SHARED_FF), lambda i: (0, 0)),
            pl.BlockSpec((BT_S, DIM), lambda i: (i, 0)),
            pl.BlockSpec((1, 1, BT_S), lambda i: (i, 0, 0)),
        ],
        out_specs=pl.BlockSpec((BT_S, DIM), lambda i: (i, 0)),
        out_shape=jax.ShapeDtypeStruct((T, DIM), jnp.float32),
    )(xf, Ws1, Ws3, Ws2, yg, gate3)

    return out.reshape(shape)
